# baseline (speedup 1.0000x reference)
import jax
import jax.numpy as jnp
from jax.experimental import pallas as pl
from jax.experimental.pallas import tpu as pltpu

N = 2048
D = 128
ALPHA = 0.2
BR = 512
NB = N // BR
NEG = -9e15


def _body(xs_ref, c0_ref, adj_ref, wgx_ref, wgh_ref, atop_ref, abot_ref,
          w_ref, b_ref, c1t_ref, c2t_ref, c3t_ref, cb1_ref, cb2_ref,
          cb3_ref, h0_ref, out_ref,
          h_s, c_s, wh_s, e1_s, e2_s, acc_s, kt_s, bias_s):
    s = pl.program_id(0)
    j = pl.program_id(1)
    f32 = jnp.float32

    @pl.when(jnp.logical_and(s == 0, j == 0))
    def _init():
        h_s[:] = jnp.broadcast_to(h0_ref[:], (N, D))
        c_s[:] = c0_ref[:]
        at0 = jnp.dot(c2t_ref[0], c3t_ref[0], preferred_element_type=f32)
        at1 = (jnp.dot(c2t_ref[0], c3t_ref[1], preferred_element_type=f32)
               + jnp.dot(c2t_ref[1], c3t_ref[0], preferred_element_type=f32))
        at2 = jnp.dot(c2t_ref[1], c3t_ref[1], preferred_element_type=f32)
        kt_s[0] = jnp.dot(c1t_ref[0], at0, preferred_element_type=f32)
        kt_s[1] = (jnp.dot(c1t_ref[1], at0, preferred_element_type=f32)
                   + jnp.dot(c1t_ref[0], at1, preferred_element_type=f32))
        kt_s[2] = (jnp.dot(c1t_ref[1], at1, preferred_element_type=f32)
                   + jnp.dot(c1t_ref[0], at2, preferred_element_type=f32))
        kt_s[3] = jnp.dot(c1t_ref[1], at2, preferred_element_type=f32)
        bias_s[:] = (cb3_ref[:]
                     + jnp.dot(cb2_ref[:], c3t_ref[0] + c3t_ref[1],
                               preferred_element_type=f32)
                     + jnp.dot(cb1_ref[:], at0 + at1 + at2,
                               preferred_element_type=f32))

    @pl.when(j == 0)
    def _phase0():
        x = xs_ref[0]
        h = h_s[:]
        for g in range(4):
            whg = (jnp.dot(x, wgx_ref[g], preferred_element_type=f32)
                   + jnp.dot(h, wgh_ref[g], preferred_element_type=f32))
            wh_s[g] = whg
            e1_s[g] = jnp.dot(whg, atop_ref[g], preferred_element_type=f32)
            e2_s[g] = jax.lax.dot_general(
                abot_ref[g], whg, (((1,), (1,)), ((), ())),
                preferred_element_type=f32)

    @pl.when(j > 0)
    def _rows():
        rs = pl.ds((j - 1) * BR, BR)
        adjf = adj_ref[0].astype(f32)
        acts = []
        for g in range(4):
            logits = e1_s[g, rs, :] + e2_s[g]
            lr = jnp.where(logits > 0, logits, ALPHA * logits)
            m = jnp.max(lr, axis=1, keepdims=True)
            p = jnp.exp(lr - m) * adjf
            denom = jnp.sum(p, axis=1, keepdims=True)
            num = jnp.dot(p, wh_s[g], preferred_element_type=f32)
            av = num / denom
            acts.append(jnp.where(av > 0, av, jnp.exp(av) - 1.0))
        fg = jax.nn.sigmoid(acts[0])
        ig = jax.nn.sigmoid(acts[1])
        ic = jnp.tanh(acts[2])
        og = jax.nn.sigmoid(acts[3])
        cn = c_s[rs, :] * fg + ig * ic
        hn = jnp.tanh(cn) * og
        c_s[rs, :] = cn
        h_s[rs, :] = hn
        ot = jax.nn.sigmoid(
            jnp.dot(hn, w_ref[:], preferred_element_type=f32) + b_ref[:])
        kmat = jnp.where(s == 0, kt_s[0],
               jnp.where(s == 1, kt_s[1],
               jnp.where(s == 2, kt_s[2], kt_s[3])))
        contrib = jnp.dot(ot, kmat, preferred_element_type=f32)

        @pl.when(s == 0)
        def _():
            acc_s[rs, :] = contrib + bias_s[:]

        @pl.when(jnp.logical_and(s > 0, s < 3))
        def _():
            acc_s[rs, :] = acc_s[rs, :] + contrib

        @pl.when(s == 3)
        def _():
            out_ref[:] = acc_s[rs, :] + contrib


def kernel(actors_inter_cat, graph_adjs, Wf, af, Wi, ai, Wc, ac, Wo, ao,
           W, b, h0, cw1, cb1, cw2, cb2, cw3, cb3):
    xs = jnp.transpose(actors_inter_cat[:, 4::5, :], (1, 0, 2))
    c0 = actors_inter_cat[:, 0, :]
    adj_sel = jnp.stack([graph_adjs[:, 0, :], graph_adjs[:, 4, :],
                         graph_adjs[:, 9, :], graph_adjs[:, 14, :]],
                        axis=0).astype(jnp.bfloat16)
    wgx = jnp.stack([Wf[:D], Wi[:D], Wc[:D], Wo[:D]])
    wgh = jnp.stack([Wf[D:], Wi[D:], Wc[D:], Wo[D:]])
    atop = jnp.stack([af[:D], ai[:D], ac[:D], ao[:D]])
    abot = jnp.stack([af[D:].T, ai[D:].T, ac[D:].T, ao[D:].T])
    c1t = jnp.transpose(cw1, (2, 1, 0))
    c2t = jnp.transpose(cw2, (2, 1, 0))
    c3t = jnp.transpose(cw3, (2, 1, 0))
    cb1r = cb1.reshape(1, D)
    cb2r = cb2.reshape(1, D)
    cb3r = cb3.reshape(1, D)

    grid = (4, NB + 1)
    full2 = lambda s, j: (0, 0)
    full3 = lambda s, j: (0, 0, 0)
    in_specs = [
        pl.BlockSpec((1, N, D), lambda s, j: (s, 0, 0)),
        pl.BlockSpec((N, D), full2),
        pl.BlockSpec((1, BR, N),
                     lambda s, j: (s, jnp.maximum(j - 1, 0), 0)),
        pl.BlockSpec((4, D, D), full3),
        pl.BlockSpec((4, D, D), full3),
        pl.BlockSpec((4, D, 1), full3),
        pl.BlockSpec((4, 1, D), full3),
        pl.BlockSpec((D, D), full2),
        pl.BlockSpec((1, D), full2),
        pl.BlockSpec((2, D, D), full3),
        pl.BlockSpec((2, D, D), full3),
        pl.BlockSpec((2, D, D), full3),
        pl.BlockSpec((1, D), full2),
        pl.BlockSpec((1, D), full2),
        pl.BlockSpec((1, D), full2),
        pl.BlockSpec((1, D), full2),
    ]
    out = pl.pallas_call(
        _body,
        grid=grid,
        in_specs=in_specs,
        out_specs=pl.BlockSpec(
            (BR, D),
            lambda s, j: (jnp.where(s == 3, jnp.maximum(j - 1, 0), 0), 0)),
        out_shape=jax.ShapeDtypeStruct((N, D), jnp.float32),
        scratch_shapes=[
            pltpu.VMEM((N, D), jnp.float32),
            pltpu.VMEM((N, D), jnp.float32),
            pltpu.VMEM((4, N, D), jnp.float32),
            pltpu.VMEM((4, N, 1), jnp.float32),
            pltpu.VMEM((4, 1, N), jnp.float32),
            pltpu.VMEM((N, D), jnp.float32),
            pltpu.VMEM((4, D, D), jnp.float32),
            pltpu.VMEM((1, D), jnp.float32),
        ],
        compiler_params=pltpu.CompilerParams(
            dimension_semantics=("arbitrary", "arbitrary")),
    )(xs, c0, adj_sel, wgx, wgh, atop, abot, W, b,
      c1t, c2t, c3t, cb1r, cb2r, cb3r, h0)
    return out


# speedup vs baseline: 1.4646x; 1.4646x over previous
import jax
import jax.numpy as jnp
from jax.experimental import pallas as pl
from jax.experimental.pallas import tpu as pltpu

N = 2048
D = 128
ALPHA = 0.2
BR = 512
NB = N // BR
NEG = -9e15


def _body(xs_ref, c0_ref, adj_ref, wgx_ref, wgh_ref, atop_ref, abot_ref,
          w_ref, b_ref, c1t_ref, c2t_ref, c3t_ref, cb1_ref, cb2_ref,
          cb3_ref, h0_ref, out_ref,
          h_s, c_s, wh_s, whbf_s, e1_s, e2_s, acc_s, kt_s, bias_s):
    s = pl.program_id(0)
    j = pl.program_id(1)
    f32 = jnp.float32

    @pl.when(jnp.logical_and(s == 0, j == 0))
    def _init():
        h_s[:] = jnp.broadcast_to(h0_ref[:], (N, D))
        c_s[:] = c0_ref[:]
        at0 = jnp.dot(c2t_ref[0], c3t_ref[0], preferred_element_type=f32)
        at1 = (jnp.dot(c2t_ref[0], c3t_ref[1], preferred_element_type=f32)
               + jnp.dot(c2t_ref[1], c3t_ref[0], preferred_element_type=f32))
        at2 = jnp.dot(c2t_ref[1], c3t_ref[1], preferred_element_type=f32)
        kt_s[0] = jnp.dot(c1t_ref[0], at0, preferred_element_type=f32)
        kt_s[1] = (jnp.dot(c1t_ref[1], at0, preferred_element_type=f32)
                   + jnp.dot(c1t_ref[0], at1, preferred_element_type=f32))
        kt_s[2] = (jnp.dot(c1t_ref[1], at1, preferred_element_type=f32)
                   + jnp.dot(c1t_ref[0], at2, preferred_element_type=f32))
        kt_s[3] = jnp.dot(c1t_ref[1], at2, preferred_element_type=f32)
        bias_s[:] = (cb3_ref[:]
                     + jnp.dot(cb2_ref[:], c3t_ref[0] + c3t_ref[1],
                               preferred_element_type=f32)
                     + jnp.dot(cb1_ref[:], at0 + at1 + at2,
                               preferred_element_type=f32))

    @pl.when(j == 0)
    def _phase0():
        x = xs_ref[0]
        h = h_s[:]
        for g in range(4):
            whg = (jnp.dot(x, wgx_ref[g], preferred_element_type=f32)
                   + jnp.dot(h, wgh_ref[g], preferred_element_type=f32))
            wh_s[g] = whg
            whbf_s[g] = whg.astype(jnp.bfloat16)
            e1_s[g] = jnp.dot(whg, atop_ref[g], preferred_element_type=f32)
            e2_s[g] = jax.lax.dot_general(
                abot_ref[g], whg, (((1,), (1,)), ((), ())),
                preferred_element_type=f32)

    @pl.when(j > 0)
    def _rows():
        rs = pl.ds((j - 1) * BR, BR)
        adjf = adj_ref[0]
        acts = []
        for g in range(4):
            logits = e1_s[g, rs, :] + e2_s[g]
            lr = jnp.where(logits > 0, logits, ALPHA * logits)
            m = jnp.max(lr, axis=1, keepdims=True)
            p = jnp.exp(lr - m) * adjf
            denom = jnp.sum(p, axis=1, keepdims=True)
            num = jnp.dot(p.astype(jnp.bfloat16), whbf_s[g],
                          preferred_element_type=f32)
            av = num / denom
            acts.append(jnp.where(av > 0, av, jnp.exp(av) - 1.0))
        fg = jax.nn.sigmoid(acts[0])
        ig = jax.nn.sigmoid(acts[1])
        ic = jnp.tanh(acts[2])
        og = jax.nn.sigmoid(acts[3])
        cn = c_s[rs, :] * fg + ig * ic
        hn = jnp.tanh(cn) * og
        c_s[rs, :] = cn
        h_s[rs, :] = hn
        ot = jax.nn.sigmoid(
            jnp.dot(hn, w_ref[:], preferred_element_type=f32) + b_ref[:])
        kmat = jnp.where(s == 0, kt_s[0],
               jnp.where(s == 1, kt_s[1],
               jnp.where(s == 2, kt_s[2], kt_s[3])))
        contrib = jnp.dot(ot, kmat, preferred_element_type=f32)

        @pl.when(s == 0)
        def _():
            acc_s[rs, :] = contrib + bias_s[:]

        @pl.when(jnp.logical_and(s > 0, s < 3))
        def _():
            acc_s[rs, :] = acc_s[rs, :] + contrib

        @pl.when(s == 3)
        def _():
            out_ref[:] = acc_s[rs, :] + contrib


def kernel(actors_inter_cat, graph_adjs, Wf, af, Wi, ai, Wc, ac, Wo, ao,
           W, b, h0, cw1, cb1, cw2, cb2, cw3, cb3):
    xs = jnp.transpose(actors_inter_cat[:, 4::5, :], (1, 0, 2))
    c0 = actors_inter_cat[:, 0, :]
    adj_sel = jnp.stack([graph_adjs[:, 0, :], graph_adjs[:, 4, :],
                         graph_adjs[:, 9, :], graph_adjs[:, 14, :]],
                        axis=0)
    wgx = jnp.stack([Wf[:D], Wi[:D], Wc[:D], Wo[:D]])
    wgh = jnp.stack([Wf[D:], Wi[D:], Wc[D:], Wo[D:]])
    atop = jnp.stack([af[:D], ai[:D], ac[:D], ao[:D]])
    abot = jnp.stack([af[D:].T, ai[D:].T, ac[D:].T, ao[D:].T])
    c1t = jnp.transpose(cw1, (2, 1, 0))
    c2t = jnp.transpose(cw2, (2, 1, 0))
    c3t = jnp.transpose(cw3, (2, 1, 0))
    cb1r = cb1.reshape(1, D)
    cb2r = cb2.reshape(1, D)
    cb3r = cb3.reshape(1, D)

    grid = (4, NB + 1)
    full2 = lambda s, j: (0, 0)
    full3 = lambda s, j: (0, 0, 0)
    in_specs = [
        pl.BlockSpec((1, N, D), lambda s, j: (s, 0, 0)),
        pl.BlockSpec((N, D), full2),
        pl.BlockSpec((1, BR, N),
                     lambda s, j: (s, jnp.maximum(j - 1, 0), 0)),
        pl.BlockSpec((4, D, D), full3),
        pl.BlockSpec((4, D, D), full3),
        pl.BlockSpec((4, D, 1), full3),
        pl.BlockSpec((4, 1, D), full3),
        pl.BlockSpec((D, D), full2),
        pl.BlockSpec((1, D), full2),
        pl.BlockSpec((2, D, D), full3),
        pl.BlockSpec((2, D, D), full3),
        pl.BlockSpec((2, D, D), full3),
        pl.BlockSpec((1, D), full2),
        pl.BlockSpec((1, D), full2),
        pl.BlockSpec((1, D), full2),
        pl.BlockSpec((1, D), full2),
    ]
    out = pl.pallas_call(
        _body,
        grid=grid,
        in_specs=in_specs,
        out_specs=pl.BlockSpec(
            (BR, D),
            lambda s, j: (jnp.where(s == 3, jnp.maximum(j - 1, 0), 0), 0)),
        out_shape=jax.ShapeDtypeStruct((N, D), jnp.float32),
        scratch_shapes=[
            pltpu.VMEM((N, D), jnp.float32),
            pltpu.VMEM((N, D), jnp.float32),
            pltpu.VMEM((4, N, D), jnp.float32),
            pltpu.VMEM((4, N, D), jnp.bfloat16),
            pltpu.VMEM((4, N, 1), jnp.float32),
            pltpu.VMEM((4, 1, N), jnp.float32),
            pltpu.VMEM((N, D), jnp.float32),
            pltpu.VMEM((4, D, D), jnp.float32),
            pltpu.VMEM((1, D), jnp.float32),
        ],
        compiler_params=pltpu.CompilerParams(
            dimension_semantics=("arbitrary", "arbitrary")),
    )(xs, c0, adj_sel, wgx, wgh, atop, abot, W, b,
      c1t, c2t, c3t, cb1r, cb2r, cb3r, h0)
    return out


# speedup vs baseline: 1.7420x; 1.1894x over previous
import jax
import jax.numpy as jnp
from jax.experimental import pallas as pl
from jax.experimental.pallas import tpu as pltpu

N = 2048
D = 128
ALPHA = 0.2
BR = 512
NB = N // BR
NEG = -9e15


def _body(xs_ref, c0_ref, adj_ref, wgx_ref, wgh_ref, atop_ref, abot_ref,
          w_ref, b_ref, c1t_ref, c2t_ref, c3t_ref, cb1_ref, cb2_ref,
          cb3_ref, h0_ref, out_ref,
          h_s, c_s, whbf_s, e1_s, e2_s, em_s, acc_s, kt_s, bias_s):
    s = pl.program_id(0)
    j = pl.program_id(1)
    f32 = jnp.float32

    @pl.when(jnp.logical_and(s == 0, j == 0))
    def _init():
        h_s[:] = jnp.broadcast_to(h0_ref[:], (N, D))
        c_s[:] = c0_ref[:]
        whbf_s[:, :, D:] = jnp.ones((4, N, D), jnp.bfloat16)
        at0 = jnp.dot(c2t_ref[0], c3t_ref[0], preferred_element_type=f32)
        at1 = (jnp.dot(c2t_ref[0], c3t_ref[1], preferred_element_type=f32)
               + jnp.dot(c2t_ref[1], c3t_ref[0], preferred_element_type=f32))
        at2 = jnp.dot(c2t_ref[1], c3t_ref[1], preferred_element_type=f32)
        kt_s[0] = jnp.dot(c1t_ref[0], at0, preferred_element_type=f32)
        kt_s[1] = (jnp.dot(c1t_ref[1], at0, preferred_element_type=f32)
                   + jnp.dot(c1t_ref[0], at1, preferred_element_type=f32))
        kt_s[2] = (jnp.dot(c1t_ref[1], at1, preferred_element_type=f32)
                   + jnp.dot(c1t_ref[0], at2, preferred_element_type=f32))
        kt_s[3] = jnp.dot(c1t_ref[1], at2, preferred_element_type=f32)
        bias_s[:] = (cb3_ref[:]
                     + jnp.dot(cb2_ref[:], c3t_ref[0] + c3t_ref[1],
                               preferred_element_type=f32)
                     + jnp.dot(cb1_ref[:], at0 + at1 + at2,
                               preferred_element_type=f32))

    @pl.when(j == 0)
    def _phase0():
        x = xs_ref[0]
        h = h_s[:]
        for g in range(4):
            whg = (jnp.dot(x, wgx_ref[g], preferred_element_type=f32)
                   + jnp.dot(h, wgh_ref[g], preferred_element_type=f32))
            whbf_s[g, :, :D] = whg.astype(jnp.bfloat16)
            e1 = jnp.dot(whg, atop_ref[g], preferred_element_type=f32)
            e1_s[g] = e1
            e2 = jax.lax.dot_general(
                abot_ref[g], whg, (((1,), (1,)), ((), ())),
                preferred_element_type=f32)
            e2_s[g] = e2
            z = e1 + jnp.max(e2)
            em_s[g] = jnp.maximum(z, ALPHA * z)

    @pl.when(j > 0)
    def _rows():
        rs = pl.ds((j - 1) * BR, BR)
        adjf = adj_ref[0]
        acts = []
        for g in range(4):
            logits = e1_s[g, rs, :] + e2_s[g]
            lr = jnp.maximum(logits, ALPHA * logits)
            p = jnp.exp(lr - em_s[g, rs, :]) * adjf
            num2 = jnp.dot(p.astype(jnp.bfloat16), whbf_s[g],
                           preferred_element_type=f32)
            av = num2[:, :D] / num2[:, D:D + 1]
            acts.append(jnp.where(av > 0, av, jnp.exp(av) - 1.0))
        fg = jax.nn.sigmoid(acts[0])
        ig = jax.nn.sigmoid(acts[1])
        ic = jnp.tanh(acts[2])
        og = jax.nn.sigmoid(acts[3])
        cn = c_s[rs, :] * fg + ig * ic
        hn = jnp.tanh(cn) * og
        c_s[rs, :] = cn
        h_s[rs, :] = hn
        ot = jax.nn.sigmoid(
            jnp.dot(hn, w_ref[:], preferred_element_type=f32) + b_ref[:])
        kmat = jnp.where(s == 0, kt_s[0],
               jnp.where(s == 1, kt_s[1],
               jnp.where(s == 2, kt_s[2], kt_s[3])))
        contrib = jnp.dot(ot, kmat, preferred_element_type=f32)

        @pl.when(s == 0)
        def _():
            acc_s[rs, :] = contrib + bias_s[:]

        @pl.when(jnp.logical_and(s > 0, s < 3))
        def _():
            acc_s[rs, :] = acc_s[rs, :] + contrib

        @pl.when(s == 3)
        def _():
            out_ref[:] = acc_s[rs, :] + contrib


def kernel(actors_inter_cat, graph_adjs, Wf, af, Wi, ai, Wc, ac, Wo, ao,
           W, b, h0, cw1, cb1, cw2, cb2, cw3, cb3):
    xs = jnp.transpose(actors_inter_cat[:, 4::5, :], (1, 0, 2))
    c0 = actors_inter_cat[:, 0, :]
    adj_sel = jnp.stack([graph_adjs[:, 0, :], graph_adjs[:, 4, :],
                         graph_adjs[:, 9, :], graph_adjs[:, 14, :]],
                        axis=0)
    wgx = jnp.stack([Wf[:D], Wi[:D], Wc[:D], Wo[:D]])
    wgh = jnp.stack([Wf[D:], Wi[D:], Wc[D:], Wo[D:]])
    atop = jnp.stack([af[:D], ai[:D], ac[:D], ao[:D]])
    abot = jnp.stack([af[D:].T, ai[D:].T, ac[D:].T, ao[D:].T])
    c1t = jnp.transpose(cw1, (2, 1, 0))
    c2t = jnp.transpose(cw2, (2, 1, 0))
    c3t = jnp.transpose(cw3, (2, 1, 0))
    cb1r = cb1.reshape(1, D)
    cb2r = cb2.reshape(1, D)
    cb3r = cb3.reshape(1, D)

    grid = (4, NB + 1)
    full2 = lambda s, j: (0, 0)
    full3 = lambda s, j: (0, 0, 0)
    in_specs = [
        pl.BlockSpec((1, N, D), lambda s, j: (s, 0, 0)),
        pl.BlockSpec((N, D), full2),
        pl.BlockSpec((1, BR, N),
                     lambda s, j: (s, jnp.maximum(j - 1, 0), 0)),
        pl.BlockSpec((4, D, D), full3),
        pl.BlockSpec((4, D, D), full3),
        pl.BlockSpec((4, D, 1), full3),
        pl.BlockSpec((4, 1, D), full3),
        pl.BlockSpec((D, D), full2),
        pl.BlockSpec((1, D), full2),
        pl.BlockSpec((2, D, D), full3),
        pl.BlockSpec((2, D, D), full3),
        pl.BlockSpec((2, D, D), full3),
        pl.BlockSpec((1, D), full2),
        pl.BlockSpec((1, D), full2),
        pl.BlockSpec((1, D), full2),
        pl.BlockSpec((1, D), full2),
    ]
    out = pl.pallas_call(
        _body,
        grid=grid,
        in_specs=in_specs,
        out_specs=pl.BlockSpec(
            (BR, D),
            lambda s, j: (jnp.where(s == 3, jnp.maximum(j - 1, 0), 0), 0)),
        out_shape=jax.ShapeDtypeStruct((N, D), jnp.float32),
        scratch_shapes=[
            pltpu.VMEM((N, D), jnp.float32),
            pltpu.VMEM((N, D), jnp.float32),
            pltpu.VMEM((4, N, 2 * D), jnp.bfloat16),
            pltpu.VMEM((4, N, 1), jnp.float32),
            pltpu.VMEM((4, 1, N), jnp.float32),
            pltpu.VMEM((4, N, 1), jnp.float32),
            pltpu.VMEM((N, D), jnp.float32),
            pltpu.VMEM((4, D, D), jnp.float32),
            pltpu.VMEM((1, D), jnp.float32),
        ],
        compiler_params=pltpu.CompilerParams(
            dimension_semantics=("arbitrary", "arbitrary")),
    )(xs, c0, adj_sel, wgx, wgh, atop, abot, W, b,
      c1t, c2t, c3t, cb1r, cb2r, cb3r, h0)
    return out


# speedup vs baseline: 1.8495x; 1.0617x over previous
import jax
import jax.numpy as jnp
from jax.experimental import pallas as pl
from jax.experimental.pallas import tpu as pltpu

N = 2048
D = 128
ALPHA = 0.2
BR = 512
NB = N // BR
LOG2E = 1.4426950408889634


def _body(xs_ref, c0_ref, adj_ref, wgx_ref, wgh_ref, atop_ref, abot_ref,
          w_ref, b_ref, c1t_ref, c2t_ref, c3t_ref, cb1_ref, cb2_ref,
          cb3_ref, h0_ref, out_ref,
          h_s, c_s, whbf_s, u1_s, v1_s, u2_s, v2_s, acc_s, kt_s, bias_s):
    s = pl.program_id(0)
    j = pl.program_id(1)
    f32 = jnp.float32

    @pl.when(jnp.logical_and(s == 0, j == 0))
    def _init():
        h_s[:] = jnp.broadcast_to(h0_ref[:], (N, D))
        c_s[:] = c0_ref[:]
        whbf_s[:, :, D:] = jnp.ones((4, N, D), jnp.bfloat16)
        at0 = jnp.dot(c2t_ref[0], c3t_ref[0], preferred_element_type=f32)
        at1 = (jnp.dot(c2t_ref[0], c3t_ref[1], preferred_element_type=f32)
               + jnp.dot(c2t_ref[1], c3t_ref[0], preferred_element_type=f32))
        at2 = jnp.dot(c2t_ref[1], c3t_ref[1], preferred_element_type=f32)
        kt_s[0] = jnp.dot(c1t_ref[0], at0, preferred_element_type=f32)
        kt_s[1] = (jnp.dot(c1t_ref[1], at0, preferred_element_type=f32)
                   + jnp.dot(c1t_ref[0], at1, preferred_element_type=f32))
        kt_s[2] = (jnp.dot(c1t_ref[1], at1, preferred_element_type=f32)
                   + jnp.dot(c1t_ref[0], at2, preferred_element_type=f32))
        kt_s[3] = jnp.dot(c1t_ref[1], at2, preferred_element_type=f32)
        bias_s[:] = (cb3_ref[:]
                     + jnp.dot(cb2_ref[:], c3t_ref[0] + c3t_ref[1],
                               preferred_element_type=f32)
                     + jnp.dot(cb1_ref[:], at0 + at1 + at2,
                               preferred_element_type=f32))

    @pl.when(j == 0)
    def _phase0():
        x = xs_ref[0]
        h = h_s[:]
        for g in range(4):
            whg = (jnp.dot(x, wgx_ref[g], preferred_element_type=f32)
                   + jnp.dot(h, wgh_ref[g], preferred_element_type=f32))
            whbf_s[g, :, :D] = whg.astype(jnp.bfloat16)
            e1 = jnp.dot(whg, atop_ref[g], preferred_element_type=f32)
            e2 = jax.lax.dot_general(
                abot_ref[g], whg, (((1,), (1,)), ((), ())),
                preferred_element_type=f32)
            z = e1 + jnp.max(e2)
            m = jnp.maximum(z, ALPHA * z)
            u1_s[g] = (e1 - m) * LOG2E
            v1_s[g] = (ALPHA * e1 - m) * LOG2E
            u2_s[g] = e2 * LOG2E
            v2_s[g] = (ALPHA * LOG2E) * e2

    @pl.when(j > 0)
    def _rows():
        rs = pl.ds((j - 1) * BR, BR)
        adjf = adj_ref[0]
        acts = []
        for g in range(4):
            p = jnp.exp2(jnp.maximum(u1_s[g, rs, :] + u2_s[g],
                                     v1_s[g, rs, :] + v2_s[g])) * adjf
            num2 = jnp.dot(p.astype(jnp.bfloat16), whbf_s[g],
                           preferred_element_type=f32)
            av = num2[:, :D] / num2[:, D:D + 1]
            acts.append(jnp.where(av > 0, av, jnp.exp(av) - 1.0))
        fg = jax.nn.sigmoid(acts[0])
        ig = jax.nn.sigmoid(acts[1])
        ic = jnp.tanh(acts[2])
        og = jax.nn.sigmoid(acts[3])
        cn = c_s[rs, :] * fg + ig * ic
        hn = jnp.tanh(cn) * og
        c_s[rs, :] = cn
        h_s[rs, :] = hn
        ot = jax.nn.sigmoid(
            jnp.dot(hn, w_ref[:], preferred_element_type=f32) + b_ref[:])
        kmat = jnp.where(s == 0, kt_s[0],
               jnp.where(s == 1, kt_s[1],
               jnp.where(s == 2, kt_s[2], kt_s[3])))
        contrib = jnp.dot(ot, kmat, preferred_element_type=f32)

        @pl.when(s == 0)
        def _():
            acc_s[rs, :] = contrib + bias_s[:]

        @pl.when(jnp.logical_and(s > 0, s < 3))
        def _():
            acc_s[rs, :] = acc_s[rs, :] + contrib

        @pl.when(s == 3)
        def _():
            out_ref[:] = acc_s[rs, :] + contrib


def kernel(actors_inter_cat, graph_adjs, Wf, af, Wi, ai, Wc, ac, Wo, ao,
           W, b, h0, cw1, cb1, cw2, cb2, cw3, cb3):
    xs = jnp.transpose(actors_inter_cat[:, 4::5, :], (1, 0, 2))
    c0 = actors_inter_cat[:, 0, :]
    adj_sel = jnp.stack([graph_adjs[:, 0, :], graph_adjs[:, 4, :],
                         graph_adjs[:, 9, :], graph_adjs[:, 14, :]],
                        axis=0)
    wgx = jnp.stack([Wf[:D], Wi[:D], Wc[:D], Wo[:D]])
    wgh = jnp.stack([Wf[D:], Wi[D:], Wc[D:], Wo[D:]])
    atop = jnp.stack([af[:D], ai[:D], ac[:D], ao[:D]])
    abot = jnp.stack([af[D:].T, ai[D:].T, ac[D:].T, ao[D:].T])
    c1t = jnp.transpose(cw1, (2, 1, 0))
    c2t = jnp.transpose(cw2, (2, 1, 0))
    c3t = jnp.transpose(cw3, (2, 1, 0))
    cb1r = cb1.reshape(1, D)
    cb2r = cb2.reshape(1, D)
    cb3r = cb3.reshape(1, D)

    grid = (4, NB + 1)
    full2 = lambda s, j: (0, 0)
    full3 = lambda s, j: (0, 0, 0)
    in_specs = [
        pl.BlockSpec((1, N, D), lambda s, j: (s, 0, 0)),
        pl.BlockSpec((N, D), full2),
        pl.BlockSpec((1, BR, N),
                     lambda s, j: (s, jnp.maximum(j - 1, 0), 0)),
        pl.BlockSpec((4, D, D), full3),
        pl.BlockSpec((4, D, D), full3),
        pl.BlockSpec((4, D, 1), full3),
        pl.BlockSpec((4, 1, D), full3),
        pl.BlockSpec((D, D), full2),
        pl.BlockSpec((1, D), full2),
        pl.BlockSpec((2, D, D), full3),
        pl.BlockSpec((2, D, D), full3),
        pl.BlockSpec((2, D, D), full3),
        pl.BlockSpec((1, D), full2),
        pl.BlockSpec((1, D), full2),
        pl.BlockSpec((1, D), full2),
        pl.BlockSpec((1, D), full2),
    ]
    out = pl.pallas_call(
        _body,
        grid=grid,
        in_specs=in_specs,
        out_specs=pl.BlockSpec(
            (BR, D),
            lambda s, j: (jnp.where(s == 3, jnp.maximum(j - 1, 0), 0), 0)),
        out_shape=jax.ShapeDtypeStruct((N, D), jnp.float32),
        scratch_shapes=[
            pltpu.VMEM((N, D), jnp.float32),
            pltpu.VMEM((N, D), jnp.float32),
            pltpu.VMEM((4, N, 2 * D), jnp.bfloat16),
            pltpu.VMEM((4, N, 1), jnp.float32),
            pltpu.VMEM((4, N, 1), jnp.float32),
            pltpu.VMEM((4, 1, N), jnp.float32),
            pltpu.VMEM((4, 1, N), jnp.float32),
            pltpu.VMEM((N, D), jnp.float32),
            pltpu.VMEM((4, D, D), jnp.float32),
            pltpu.VMEM((1, D), jnp.float32),
        ],
        compiler_params=pltpu.CompilerParams(
            dimension_semantics=("arbitrary", "arbitrary")),
    )(xs, c0, adj_sel, wgx, wgh, atop, abot, W, b,
      c1t, c2t, c3t, cb1r, cb2r, cb3r, h0)
    return out


# speedup vs baseline: 1.8524x; 1.0016x over previous
import jax
import jax.numpy as jnp
from jax.experimental import pallas as pl
from jax.experimental.pallas import tpu as pltpu

N = 2048
D = 128
ALPHA = 0.2
BR = 1024
NB = N // BR
LOG2E = 1.4426950408889634


def _body(xs_ref, c0_ref, adj_ref, wgx_ref, wgh_ref, atop_ref, abot_ref,
          w_ref, b_ref, c1t_ref, c2t_ref, c3t_ref, cb1_ref, cb2_ref,
          cb3_ref, h0_ref, out_ref,
          h_s, c_s, whbf_s, u1_s, v1_s, u2_s, v2_s, acc_s, kt_s, bias_s):
    s = pl.program_id(0)
    j = pl.program_id(1)
    f32 = jnp.float32

    @pl.when(jnp.logical_and(s == 0, j == 0))
    def _init():
        h_s[:] = jnp.broadcast_to(h0_ref[:], (N, D))
        c_s[:] = c0_ref[:]
        whbf_s[:, :, D:] = jnp.ones((4, N, D), jnp.bfloat16)
        at0 = jnp.dot(c2t_ref[0], c3t_ref[0], preferred_element_type=f32)
        at1 = (jnp.dot(c2t_ref[0], c3t_ref[1], preferred_element_type=f32)
               + jnp.dot(c2t_ref[1], c3t_ref[0], preferred_element_type=f32))
        at2 = jnp.dot(c2t_ref[1], c3t_ref[1], preferred_element_type=f32)
        kt_s[0] = jnp.dot(c1t_ref[0], at0, preferred_element_type=f32)
        kt_s[1] = (jnp.dot(c1t_ref[1], at0, preferred_element_type=f32)
                   + jnp.dot(c1t_ref[0], at1, preferred_element_type=f32))
        kt_s[2] = (jnp.dot(c1t_ref[1], at1, preferred_element_type=f32)
                   + jnp.dot(c1t_ref[0], at2, preferred_element_type=f32))
        kt_s[3] = jnp.dot(c1t_ref[1], at2, preferred_element_type=f32)
        bias_s[:] = (cb3_ref[:]
                     + jnp.dot(cb2_ref[:], c3t_ref[0] + c3t_ref[1],
                               preferred_element_type=f32)
                     + jnp.dot(cb1_ref[:], at0 + at1 + at2,
                               preferred_element_type=f32))

    @pl.when(j == 0)
    def _phase0():
        x = xs_ref[0]
        h = h_s[:]
        for g in range(4):
            whg = (jnp.dot(x, wgx_ref[g], preferred_element_type=f32)
                   + jnp.dot(h, wgh_ref[g], preferred_element_type=f32))
            whbf_s[g, :, :D] = whg.astype(jnp.bfloat16)
            e1 = jnp.dot(whg, atop_ref[g], preferred_element_type=f32)
            e2 = jax.lax.dot_general(
                abot_ref[g], whg, (((1,), (1,)), ((), ())),
                preferred_element_type=f32)
            z = e1 + jnp.max(e2)
            m = jnp.maximum(z, ALPHA * z)
            u1_s[g] = (e1 - m) * LOG2E
            v1_s[g] = (ALPHA * e1 - m) * LOG2E
            u2_s[g] = e2 * LOG2E
            v2_s[g] = (ALPHA * LOG2E) * e2

    @pl.when(j > 0)
    def _rows():
        rs = pl.ds((j - 1) * BR, BR)
        adjf = adj_ref[0]
        acts = []
        for g in range(4):
            p = jnp.exp2(jnp.maximum(u1_s[g, rs, :] + u2_s[g],
                                     v1_s[g, rs, :] + v2_s[g])) * adjf
            num2 = jnp.dot(p.astype(jnp.bfloat16), whbf_s[g],
                           preferred_element_type=f32)
            av = num2[:, :D] / num2[:, D:D + 1]
            acts.append(jnp.where(av > 0, av, jnp.exp(av) - 1.0))
        fg = jax.nn.sigmoid(acts[0])
        ig = jax.nn.sigmoid(acts[1])
        ic = jnp.tanh(acts[2])
        og = jax.nn.sigmoid(acts[3])
        cn = c_s[rs, :] * fg + ig * ic
        hn = jnp.tanh(cn) * og
        c_s[rs, :] = cn
        h_s[rs, :] = hn
        ot = jax.nn.sigmoid(
            jnp.dot(hn, w_ref[:], preferred_element_type=f32) + b_ref[:])
        kmat = jnp.where(s == 0, kt_s[0],
               jnp.where(s == 1, kt_s[1],
               jnp.where(s == 2, kt_s[2], kt_s[3])))
        contrib = jnp.dot(ot, kmat, preferred_element_type=f32)

        @pl.when(s == 0)
        def _():
            acc_s[rs, :] = contrib + bias_s[:]

        @pl.when(jnp.logical_and(s > 0, s < 3))
        def _():
            acc_s[rs, :] = acc_s[rs, :] + contrib

        @pl.when(s == 3)
        def _():
            out_ref[:] = acc_s[rs, :] + contrib


def kernel(actors_inter_cat, graph_adjs, Wf, af, Wi, ai, Wc, ac, Wo, ao,
           W, b, h0, cw1, cb1, cw2, cb2, cw3, cb3):
    xs = jnp.transpose(actors_inter_cat[:, 4::5, :], (1, 0, 2))
    c0 = actors_inter_cat[:, 0, :]
    adj_sel = jnp.stack([graph_adjs[:, 0, :], graph_adjs[:, 4, :],
                         graph_adjs[:, 9, :], graph_adjs[:, 14, :]],
                        axis=0)
    wgx = jnp.stack([Wf[:D], Wi[:D], Wc[:D], Wo[:D]])
    wgh = jnp.stack([Wf[D:], Wi[D:], Wc[D:], Wo[D:]])
    atop = jnp.stack([af[:D], ai[:D], ac[:D], ao[:D]])
    abot = jnp.stack([af[D:].T, ai[D:].T, ac[D:].T, ao[D:].T])
    c1t = jnp.transpose(cw1, (2, 1, 0))
    c2t = jnp.transpose(cw2, (2, 1, 0))
    c3t = jnp.transpose(cw3, (2, 1, 0))
    cb1r = cb1.reshape(1, D)
    cb2r = cb2.reshape(1, D)
    cb3r = cb3.reshape(1, D)

    grid = (4, NB + 1)
    full2 = lambda s, j: (0, 0)
    full3 = lambda s, j: (0, 0, 0)
    in_specs = [
        pl.BlockSpec((1, N, D), lambda s, j: (s, 0, 0)),
        pl.BlockSpec((N, D), full2),
        pl.BlockSpec((1, BR, N),
                     lambda s, j: (s, jnp.maximum(j - 1, 0), 0)),
        pl.BlockSpec((4, D, D), full3),
        pl.BlockSpec((4, D, D), full3),
        pl.BlockSpec((4, D, 1), full3),
        pl.BlockSpec((4, 1, D), full3),
        pl.BlockSpec((D, D), full2),
        pl.BlockSpec((1, D), full2),
        pl.BlockSpec((2, D, D), full3),
        pl.BlockSpec((2, D, D), full3),
        pl.BlockSpec((2, D, D), full3),
        pl.BlockSpec((1, D), full2),
        pl.BlockSpec((1, D), full2),
        pl.BlockSpec((1, D), full2),
        pl.BlockSpec((1, D), full2),
    ]
    out = pl.pallas_call(
        _body,
        grid=grid,
        in_specs=in_specs,
        out_specs=pl.BlockSpec(
            (BR, D),
            lambda s, j: (jnp.where(s == 3, jnp.maximum(j - 1, 0), 0), 0)),
        out_shape=jax.ShapeDtypeStruct((N, D), jnp.float32),
        scratch_shapes=[
            pltpu.VMEM((N, D), jnp.float32),
            pltpu.VMEM((N, D), jnp.float32),
            pltpu.VMEM((4, N, 2 * D), jnp.bfloat16),
            pltpu.VMEM((4, N, 1), jnp.float32),
            pltpu.VMEM((4, N, 1), jnp.float32),
            pltpu.VMEM((4, 1, N), jnp.float32),
            pltpu.VMEM((4, 1, N), jnp.float32),
            pltpu.VMEM((N, D), jnp.float32),
            pltpu.VMEM((4, D, D), jnp.float32),
            pltpu.VMEM((1, D), jnp.float32),
        ],
        compiler_params=pltpu.CompilerParams(
            dimension_semantics=("arbitrary", "arbitrary")),
    )(xs, c0, adj_sel, wgx, wgh, atop, abot, W, b,
      c1t, c2t, c3t, cb1r, cb2r, cb3r, h0)
    return out


# speedup vs baseline: 1.9351x; 1.0446x over previous
import jax
import jax.numpy as jnp
from jax.experimental import pallas as pl
from jax.experimental.pallas import tpu as pltpu

N = 2048
D = 128
ALPHA = 0.2
BR = 1024
NB = N // BR
LOG2E = 1.4426950408889634


def _body(xs_ref, c0_ref, adj_ref, wgx_ref, wgh_ref, atop_ref, abot_ref,
          w_ref, b_ref, c1t_ref, c2t_ref, c3t_ref, cb1_ref, cb2_ref,
          cb3_ref, h0_ref, out_ref,
          h_s, c_s, whbf_s, u1_s, v1_s, u2_s, v2_s, acc_s, kt_s, bias_s):
    s = pl.program_id(0)
    j = pl.program_id(1)
    f32 = jnp.float32

    @pl.when(jnp.logical_and(s == 0, j == 0))
    def _init():
        h_s[:] = jnp.broadcast_to(h0_ref[:], (N, D))
        c_s[:] = c0_ref[:]
        whbf_s[:, :, D:] = jnp.ones((4, N, D), jnp.bfloat16)
        at0 = jnp.dot(c2t_ref[0], c3t_ref[0], preferred_element_type=f32)
        at1 = (jnp.dot(c2t_ref[0], c3t_ref[1], preferred_element_type=f32)
               + jnp.dot(c2t_ref[1], c3t_ref[0], preferred_element_type=f32))
        at2 = jnp.dot(c2t_ref[1], c3t_ref[1], preferred_element_type=f32)
        kt_s[0] = jnp.dot(c1t_ref[0], at0, preferred_element_type=f32)
        kt_s[1] = (jnp.dot(c1t_ref[1], at0, preferred_element_type=f32)
                   + jnp.dot(c1t_ref[0], at1, preferred_element_type=f32))
        kt_s[2] = (jnp.dot(c1t_ref[1], at1, preferred_element_type=f32)
                   + jnp.dot(c1t_ref[0], at2, preferred_element_type=f32))
        kt_s[3] = jnp.dot(c1t_ref[1], at2, preferred_element_type=f32)
        bias_s[:] = (cb3_ref[:]
                     + jnp.dot(cb2_ref[:], c3t_ref[0] + c3t_ref[1],
                               preferred_element_type=f32)
                     + jnp.dot(cb1_ref[:], at0 + at1 + at2,
                               preferred_element_type=f32))

    @pl.when(j == 0)
    def _phase0():
        x = xs_ref[0]
        h = h_s[:]
        for g in range(4):
            whg = (jnp.dot(x, wgx_ref[g], preferred_element_type=f32)
                   + jnp.dot(h, wgh_ref[g], preferred_element_type=f32))
            whbf_s[g, :, :D] = whg.astype(jnp.bfloat16)
            e1 = jnp.dot(whg, atop_ref[g], preferred_element_type=f32)
            e2 = jax.lax.dot_general(
                abot_ref[g], whg, (((1,), (1,)), ((), ())),
                preferred_element_type=f32)
            z = e1 + jnp.max(e2)
            m = jnp.maximum(z, ALPHA * z)
            u1_s[g] = ((e1 - m) * LOG2E).astype(jnp.bfloat16)
            v1_s[g] = ((ALPHA * e1 - m) * LOG2E).astype(jnp.bfloat16)
            u2_s[g] = (e2 * LOG2E).astype(jnp.bfloat16)
            v2_s[g] = ((ALPHA * LOG2E) * e2).astype(jnp.bfloat16)

    @pl.when(j > 0)
    def _rows():
        rs = pl.ds((j - 1) * BR, BR)
        adjbf = adj_ref[0].astype(jnp.bfloat16)
        acts = []
        for g in range(4):
            p = jnp.exp2(jnp.maximum(u1_s[g, rs, :] + u2_s[g],
                                     v1_s[g, rs, :] + v2_s[g])) * adjbf
            num2 = jnp.dot(p, whbf_s[g], preferred_element_type=f32)
            av = num2[:, :D] / num2[:, D:D + 1]
            acts.append(jnp.where(av > 0, av, jnp.exp(av) - 1.0))
        fg = jax.nn.sigmoid(acts[0])
        ig = jax.nn.sigmoid(acts[1])
        ic = jnp.tanh(acts[2])
        og = jax.nn.sigmoid(acts[3])
        cn = c_s[rs, :] * fg + ig * ic
        hn = jnp.tanh(cn) * og
        c_s[rs, :] = cn
        h_s[rs, :] = hn
        ot = jax.nn.sigmoid(
            jnp.dot(hn, w_ref[:], preferred_element_type=f32) + b_ref[:])
        kmat = jnp.where(s == 0, kt_s[0],
               jnp.where(s == 1, kt_s[1],
               jnp.where(s == 2, kt_s[2], kt_s[3])))
        contrib = jnp.dot(ot, kmat, preferred_element_type=f32)

        @pl.when(s == 0)
        def _():
            acc_s[rs, :] = contrib + bias_s[:]

        @pl.when(jnp.logical_and(s > 0, s < 3))
        def _():
            acc_s[rs, :] = acc_s[rs, :] + contrib

        @pl.when(s == 3)
        def _():
            out_ref[:] = acc_s[rs, :] + contrib


def kernel(actors_inter_cat, graph_adjs, Wf, af, Wi, ai, Wc, ac, Wo, ao,
           W, b, h0, cw1, cb1, cw2, cb2, cw3, cb3):
    xs = jnp.transpose(actors_inter_cat[:, 4::5, :], (1, 0, 2))
    c0 = actors_inter_cat[:, 0, :]
    adj_sel = jnp.stack([graph_adjs[:, 0, :], graph_adjs[:, 4, :],
                         graph_adjs[:, 9, :], graph_adjs[:, 14, :]],
                        axis=0)
    wgx = jnp.stack([Wf[:D], Wi[:D], Wc[:D], Wo[:D]])
    wgh = jnp.stack([Wf[D:], Wi[D:], Wc[D:], Wo[D:]])
    atop = jnp.stack([af[:D], ai[:D], ac[:D], ao[:D]])
    abot = jnp.stack([af[D:].T, ai[D:].T, ac[D:].T, ao[D:].T])
    c1t = jnp.transpose(cw1, (2, 1, 0))
    c2t = jnp.transpose(cw2, (2, 1, 0))
    c3t = jnp.transpose(cw3, (2, 1, 0))
    cb1r = cb1.reshape(1, D)
    cb2r = cb2.reshape(1, D)
    cb3r = cb3.reshape(1, D)

    grid = (4, NB + 1)
    full2 = lambda s, j: (0, 0)
    full3 = lambda s, j: (0, 0, 0)
    in_specs = [
        pl.BlockSpec((1, N, D), lambda s, j: (s, 0, 0)),
        pl.BlockSpec((N, D), full2),
        pl.BlockSpec((1, BR, N),
                     lambda s, j: (s, jnp.maximum(j - 1, 0), 0)),
        pl.BlockSpec((4, D, D), full3),
        pl.BlockSpec((4, D, D), full3),
        pl.BlockSpec((4, D, 1), full3),
        pl.BlockSpec((4, 1, D), full3),
        pl.BlockSpec((D, D), full2),
        pl.BlockSpec((1, D), full2),
        pl.BlockSpec((2, D, D), full3),
        pl.BlockSpec((2, D, D), full3),
        pl.BlockSpec((2, D, D), full3),
        pl.BlockSpec((1, D), full2),
        pl.BlockSpec((1, D), full2),
        pl.BlockSpec((1, D), full2),
        pl.BlockSpec((1, D), full2),
    ]
    out = pl.pallas_call(
        _body,
        grid=grid,
        in_specs=in_specs,
        out_specs=pl.BlockSpec(
            (BR, D),
            lambda s, j: (jnp.where(s == 3, jnp.maximum(j - 1, 0), 0), 0)),
        out_shape=jax.ShapeDtypeStruct((N, D), jnp.float32),
        scratch_shapes=[
            pltpu.VMEM((N, D), jnp.float32),
            pltpu.VMEM((N, D), jnp.float32),
            pltpu.VMEM((4, N, 2 * D), jnp.bfloat16),
            pltpu.VMEM((4, N, 1), jnp.bfloat16),
            pltpu.VMEM((4, N, 1), jnp.bfloat16),
            pltpu.VMEM((4, 1, N), jnp.bfloat16),
            pltpu.VMEM((4, 1, N), jnp.bfloat16),
            pltpu.VMEM((N, D), jnp.float32),
            pltpu.VMEM((4, D, D), jnp.float32),
            pltpu.VMEM((1, D), jnp.float32),
        ],
        compiler_params=pltpu.CompilerParams(
            dimension_semantics=("arbitrary", "arbitrary")),
    )(xs, c0, adj_sel, wgx, wgh, atop, abot, W, b,
      c1t, c2t, c3t, cb1r, cb2r, cb3r, h0)
    return out
